# Initial kernel scaffold; baseline (speedup 1.0000x reference)
import jax
import jax.numpy as jnp
from jax.experimental import pallas as pl


def kernel(x, edge_index, W1_root, W1_nbr, b1, W2_root, W2_nbr, b2):
    raise NotImplementedError("write your pallas kernel here")


# speedup vs baseline: 7.0242x; 7.0242x over previous
import functools

import jax
import jax.numpy as jnp
from jax import lax
from jax.experimental import pallas as pl
from jax.experimental.pallas import tpu as pltpu
from jax.experimental.pallas import tpu_sc as plsc

N = 10000
E = 320000
D = 128
NC = 2
NS = 16
NW = NC * NS
EPT = E // NW
CH = 80
NCHUNK = EPT // CH
NPAD = 10240
RPT = NPAD // NS

_mesh = plsc.VectorSubcoreMesh(core_axis_name="c", subcore_axis_name="s")


@functools.partial(
    pl.kernel,
    mesh=_mesh,
    out_type=jax.ShapeDtypeStruct((NC * NPAD, D), jnp.float32),
    scratch_types=[
        pltpu.VMEM((NCHUNK, CH), jnp.int32),
        pltpu.VMEM((NCHUNK, CH), jnp.int32),
        pltpu.VMEM((CH, D), jnp.float32),
        pltpu.VMEM_SHARED((NPAD, D), jnp.float32),
    ],
)
def _sc_aggregate(y_hbm, src_hbm, dst_hbm, zeros_hbm, out_hbm,
                  src_v, dst_v, rows_v, acc):
    cid = lax.axis_index("c")
    sid = lax.axis_index("s")
    wid = sid * NC + cid

    pltpu.sync_copy(src_hbm.at[wid], src_v)
    pltpu.sync_copy(dst_hbm.at[wid], dst_v)

    pltpu.sync_copy(zeros_hbm, acc.at[pl.ds(sid * RPT, RPT)])
    plsc.subcore_barrier()

    def body(c, carry):
        pltpu.sync_copy(y_hbm.at[src_v.at[c]], rows_v)
        pltpu.sync_copy(rows_v, acc.at[dst_v.at[c]], add=True)
        return carry

    lax.fori_loop(0, NCHUNK, body, 0)

    plsc.subcore_barrier()
    pltpu.sync_copy(acc.at[pl.ds(sid * RPT, RPT)],
                    out_hbm.at[pl.ds(cid * NPAD + sid * RPT, RPT)])


def _mm_body(x_ref, w_ref, o_ref):
    o_ref[...] = jnp.dot(x_ref[...], w_ref[...],
                         preferred_element_type=jnp.float32,
                         precision=lax.Precision.HIGHEST)


_mm = pl.pallas_call(
    _mm_body,
    out_shape=jax.ShapeDtypeStruct((N, D), jnp.float32),
)


def _combine1_body(x_ref, wr_ref, b_ref, p_ref, wn2_ref, h_ref, y2_ref):
    h = jnp.dot(x_ref[...], wr_ref[...], preferred_element_type=jnp.float32,
                precision=lax.Precision.HIGHEST)
    h = h + p_ref[0, :N] + p_ref[1, :N] + b_ref[...]
    h = jnp.maximum(h, 0.0)
    h_ref[...] = h
    y2_ref[...] = jnp.dot(h, wn2_ref[...], preferred_element_type=jnp.float32,
                          precision=lax.Precision.HIGHEST)


_combine1 = pl.pallas_call(
    _combine1_body,
    out_shape=(jax.ShapeDtypeStruct((N, D), jnp.float32),
               jax.ShapeDtypeStruct((N, D), jnp.float32)),
)


def _combine2_body(h_ref, wr_ref, b_ref, p_ref, o_ref):
    o = jnp.dot(h_ref[...], wr_ref[...], preferred_element_type=jnp.float32,
                precision=lax.Precision.HIGHEST)
    o_ref[...] = o + p_ref[0, :N] + p_ref[1, :N] + b_ref[...]


_combine2 = pl.pallas_call(
    _combine2_body,
    out_shape=jax.ShapeDtypeStruct((N, D), jnp.float32),
)


def kernel(x, edge_index, W1_root, W1_nbr, b1, W2_root, W2_nbr, b2):
    src = edge_index[0].astype(jnp.int32).reshape(NW, NCHUNK, CH)
    dst = edge_index[1].astype(jnp.int32).reshape(NW, NCHUNK, CH)
    zeros = jnp.zeros((RPT, D), jnp.float32)
    b1r = b1.reshape(1, D)
    b2r = b2.reshape(1, D)

    y1 = _mm(x, W1_nbr)
    p1 = _sc_aggregate(y1, src, dst, zeros).reshape(NC, NPAD, D)
    h, y2 = _combine1(x, W1_root, b1r, p1, W2_nbr)
    p2 = _sc_aggregate(y2, src, dst, zeros).reshape(NC, NPAD, D)
    out = _combine2(h, W2_root, b2r, p2)
    return out


# speedup vs baseline: 11.2930x; 1.6077x over previous
import functools

import jax
import jax.numpy as jnp
from jax import lax
from jax.experimental import pallas as pl
from jax.experimental.pallas import tpu as pltpu
from jax.experimental.pallas import tpu_sc as plsc

N = 10000
E = 320000
D = 128
DH = D // 2
NC = 2
NS = 16
EPT = E // NS
CH = 80
NCHUNK = EPT // CH
NBUF = 4
NPAD = 10112
RPT = NPAD // NS

_mesh = plsc.VectorSubcoreMesh(core_axis_name="c", subcore_axis_name="s")


@functools.partial(
    pl.kernel,
    mesh=_mesh,
    out_type=jax.ShapeDtypeStruct((NC * NPAD, DH), jnp.float32),
    compiler_params=pltpu.CompilerParams(use_tc_tiling_on_sc=False),
    scratch_types=[
        pltpu.VMEM((NCHUNK, CH), jnp.int32),
        pltpu.VMEM((NCHUNK, CH), jnp.int32),
        pltpu.VMEM((CH, DH), jnp.float32),
        pltpu.VMEM((CH, DH), jnp.float32),
        pltpu.VMEM((CH, DH), jnp.float32),
        pltpu.VMEM((CH, DH), jnp.float32),
        pltpu.VMEM_SHARED((NPAD, DH), jnp.float32),
        pltpu.SemaphoreType.DMA,
        pltpu.SemaphoreType.DMA,
        pltpu.SemaphoreType.DMA,
        pltpu.SemaphoreType.DMA,
    ],
)
def _sc_aggregate(y_hbm, src_hbm, dst_hbm, zeros_hbm, out_hbm,
                  src_v, dst_v, buf0, buf1, buf2, buf3, acc,
                  sem0, sem1, sem2, sem3):
    cid = lax.axis_index("c")
    sid = lax.axis_index("s")
    bufs = (buf0, buf1, buf2, buf3)
    sems = (sem0, sem1, sem2, sem3)

    pltpu.sync_copy(src_hbm.at[cid * NS + sid], src_v)
    pltpu.sync_copy(dst_hbm.at[sid], dst_v)

    for b in range(NBUF):
        pltpu.async_copy(y_hbm.at[src_v.at[b]], bufs[b], sems[b])

    pltpu.sync_copy(zeros_hbm, acc.at[pl.ds(sid * RPT, RPT)])
    plsc.subcore_barrier()

    def step(c, b):
        pltpu.make_async_copy(y_hbm.at[pl.ds(0, CH)], bufs[b], sems[b]).wait()
        pltpu.sync_copy(bufs[b], acc.at[dst_v.at[c]], add=True)

        @pl.when(c + NBUF < NCHUNK)
        def _():
            pltpu.async_copy(y_hbm.at[src_v.at[c + NBUF]], bufs[b], sems[b])

    def outer(i, carry):
        for b in range(NBUF):
            step(i * NBUF + b, b)
        return carry

    lax.fori_loop(0, NCHUNK // NBUF, outer, 0)
    for t in range(NCHUNK - NCHUNK // NBUF * NBUF):
        step(NCHUNK // NBUF * NBUF + t, t)

    plsc.subcore_barrier()
    pltpu.sync_copy(acc.at[pl.ds(sid * RPT, RPT)],
                    out_hbm.at[pl.ds(cid * NPAD + sid * RPT, RPT)])


def _mm_body(x_ref, w_ref, o_ref):
    y = jnp.dot(x_ref[...], w_ref[...], preferred_element_type=jnp.float32,
                precision=lax.Precision.HIGHEST)
    o_ref[0:N] = y[:, 0:DH]
    o_ref[N:2 * N] = y[:, DH:D]


_mm = pl.pallas_call(
    _mm_body,
    out_shape=jax.ShapeDtypeStruct((2 * N, DH), jnp.float32),
)


def _combine1_body(x_ref, wr_ref, b_ref, p_ref, wn2_ref, h_ref, y2_ref):
    h = jnp.dot(x_ref[...], wr_ref[...], preferred_element_type=jnp.float32,
                precision=lax.Precision.HIGHEST)
    agg = jnp.concatenate([p_ref[0, :N], p_ref[1, :N]], axis=1)
    h = jnp.maximum(h + agg + b_ref[...], 0.0)
    h_ref[...] = h
    y2 = jnp.dot(h, wn2_ref[...], preferred_element_type=jnp.float32,
                 precision=lax.Precision.HIGHEST)
    y2_ref[0:N] = y2[:, 0:DH]
    y2_ref[N:2 * N] = y2[:, DH:D]


_combine1 = pl.pallas_call(
    _combine1_body,
    out_shape=(jax.ShapeDtypeStruct((N, D), jnp.float32),
               jax.ShapeDtypeStruct((2 * N, DH), jnp.float32)),
)


def _combine2_body(h_ref, wr_ref, b_ref, p_ref, o_ref):
    o = jnp.dot(h_ref[...], wr_ref[...], preferred_element_type=jnp.float32,
                precision=lax.Precision.HIGHEST)
    agg = jnp.concatenate([p_ref[0, :N], p_ref[1, :N]], axis=1)
    o_ref[...] = o + agg + b_ref[...]


_combine2 = pl.pallas_call(
    _combine2_body,
    out_shape=jax.ShapeDtypeStruct((N, D), jnp.float32),
)


def kernel(x, edge_index, W1_root, W1_nbr, b1, W2_root, W2_nbr, b2):
    src0 = edge_index[0].astype(jnp.int32).reshape(1, NS, NCHUNK, CH)
    src = jnp.concatenate([src0, src0 + N], axis=0).reshape(NC * NS, NCHUNK, CH)
    dst = edge_index[1].astype(jnp.int32).reshape(NS, NCHUNK, CH)
    zeros = jnp.zeros((RPT, DH), jnp.float32)
    b1r = b1.reshape(1, D)
    b2r = b2.reshape(1, D)

    y1 = _mm(x, W1_nbr)
    p1 = _sc_aggregate(y1, src, dst, zeros).reshape(NC, NPAD, DH)
    h, y2 = _combine1(x, W1_root, b1r, p1, W2_nbr)
    p2 = _sc_aggregate(y2, src, dst, zeros).reshape(NC, NPAD, DH)
    out = _combine2(h, W2_root, b2r, p2)
    return out
